# baseline (speedup 1.0000x reference)
import functools

import jax
import jax.numpy as jnp
from jax import lax
from jax.experimental import pallas as pl
from jax.experimental.pallas import tpu as pltpu
from jax.experimental.pallas import tpu_sc as plsc

_N_NODES = 100000
_D = 128
_G = 64

_NC = 2
_NS = 16
_NW = _NC * _NS

_CHUNK = 80
_NCHUNKS = _N_NODES // _CHUNK
_CPW = _NCHUNKS // _NW
_EXTRA = _NCHUNKS - _CPW * _NW
_MAXC = _CPW + 1


def _segment_sum_sc(seg_ids, h, zacc):
  mesh = plsc.VectorSubcoreMesh(core_axis_name="c", subcore_axis_name="s")

  @functools.partial(
      pl.kernel,
      out_type=jax.ShapeDtypeStruct((_NW, _G, _D), jnp.float32),
      mesh=mesh,
      scratch_types=[
          pltpu.VMEM((_CHUNK,), jnp.int32),
          pltpu.VMEM((_CHUNK,), jnp.int32),
          pltpu.VMEM((_CHUNK,), jnp.int32),
          pltpu.VMEM((_CHUNK, _D), jnp.float32),
          pltpu.VMEM((_CHUNK, _D), jnp.float32),
          pltpu.VMEM((_CHUNK, _D), jnp.float32),
          pltpu.VMEM((_G, _D), jnp.float32),
          pltpu.VMEM_SHARED((_NS, _G, _D), jnp.float32),
          pltpu.SemaphoreType.DMA,
          pltpu.SemaphoreType.DMA,
          pltpu.SemaphoreType.DMA,
          pltpu.SemaphoreType.DMA,
          pltpu.SemaphoreType.DMA,
          pltpu.SemaphoreType.DMA,
          pltpu.SemaphoreType.DMA,
          pltpu.SemaphoreType.DMA,
          pltpu.SemaphoreType.DMA,
      ],
  )
  def k(seg_hbm, h_hbm, z_hbm, out_hbm,
        i0, i1, i2, hb0, hb1, hb2, obuf_v, acc_sh,
        is0, is1, is2, hs0, hs1, hs2, ss0, ss1, ss2):
    idx = (i0, i1, i2)
    hbuf = (hb0, hb1, hb2)
    isem = (is0, is1, is2)
    hsem = (hs0, hs1, hs2)
    ssem = (ss0, ss1, ss2)

    cid = lax.axis_index("c")
    sid = lax.axis_index("s")
    wid = sid * _NC + cid
    nmine = jnp.where(wid < _EXTRA, _CPW + 1, _CPW)
    start = wid * _CPW + jnp.minimum(wid, _EXTRA)

    def g_desc(j, b):
      row0 = (start + j) * _CHUNK
      return (pltpu.make_async_copy(seg_hbm.at[pl.ds(row0, _CHUNK)],
                                    idx[b], isem[b]),
              pltpu.make_async_copy(h_hbm.at[pl.ds(row0, _CHUNK)],
                                    hbuf[b], hsem[b]))

    def s_desc(b):
      return pltpu.make_async_copy(hbuf[b], acc_sh.at[sid].at[idx[b]], ssem[b])

    pltpu.sync_copy(z_hbm, acc_sh.at[sid])

    for j in (0, 1):
      di, dh = g_desc(j, j % 3)
      di.start()
      dh.start()

    for j in range(_MAXC):
      b = j % 3
      di, dh = g_desc(j, b)

      @pl.when(j < nmine)
      def _(di=di, dh=dh, b=b):
        di.wait()
        dh.wait()
        s_desc(b).start(add=True)

      if j == 0:
        di2, dh2 = g_desc(2, 2)
        di2.start()
        dh2.start()
      else:
        bp = (j - 1) % 3
        di2, dh2 = g_desc(j + 2, (j + 2) % 3)

        @pl.when(j + 2 < nmine)
        def _(bp=bp, di2=di2, dh2=dh2):
          s_desc(bp).wait()
          di2.start()
          dh2.start()

    for b in range(3):
      s_desc(b).wait()

    pltpu.sync_copy(acc_sh.at[sid], obuf_v)
    pltpu.sync_copy(obuf_v, out_hbm.at[wid])

  return k(seg_ids, h, zacc)


def _readout_tc(partials, W0, b0):
  def mm(p_ref, w_ref, b_ref, o_ref):
    pooled = jnp.sum(p_ref[...], axis=0)
    o_ref[...] = lax.dot_general(
        pooled, w_ref[...], (((1,), (1,)), ((), ())),
        preferred_element_type=jnp.float32) + b_ref[...]

  return pl.pallas_call(
      mm,
      out_shape=jax.ShapeDtypeStruct((_G, _D), jnp.float32),
  )(partials, W0, b0.reshape(1, _D))


def kernel(segment_ids, h, W0, b0):
  zacc = jnp.zeros((_G, _D), jnp.float32)
  partials = _segment_sum_sc(segment_ids, h, zacc)
  return _readout_tc(partials, W0, b0)


# speedup vs baseline: 1.3912x; 1.3912x over previous
import functools

import jax
import jax.numpy as jnp
from jax import lax
from jax.experimental import pallas as pl
from jax.experimental.pallas import tpu as pltpu
from jax.experimental.pallas import tpu_sc as plsc

_N_NODES = 100000
_D = 128
_G = 64
_K = 8

_NC = 2
_NS = 16
_NW = _NC * _NS
_L = 16

_CHUNK = 80
_NCHUNKS = _N_NODES // _CHUNK
_CPW = _NCHUNKS // _NW
_EXTRA = _NCHUNKS - _CPW * _NW
_MAXC = _CPW + 1


def _segment_sum_sc(seg_ids, h, zacc):
  mesh = plsc.VectorSubcoreMesh(core_axis_name="c", subcore_axis_name="s")

  @functools.partial(
      pl.kernel,
      out_type=jax.ShapeDtypeStruct((_NC, _G * _K, _D), jnp.float32),
      mesh=mesh,
      scratch_types=[
          pltpu.VMEM((_CHUNK,), jnp.int32),
          pltpu.VMEM((_CHUNK,), jnp.int32),
          pltpu.VMEM((_CHUNK,), jnp.int32),
          pltpu.VMEM((_CHUNK, _D), jnp.float32),
          pltpu.VMEM((_CHUNK, _D), jnp.float32),
          pltpu.VMEM((_CHUNK, _D), jnp.float32),
          pltpu.VMEM((_G * _K, _D), jnp.float32),
          pltpu.VMEM_SHARED((_G * _K, _D), jnp.float32),
          pltpu.SemaphoreType.DMA,
          pltpu.SemaphoreType.DMA,
          pltpu.SemaphoreType.DMA,
          pltpu.SemaphoreType.DMA,
          pltpu.SemaphoreType.DMA,
          pltpu.SemaphoreType.DMA,
          pltpu.SemaphoreType.DMA,
          pltpu.SemaphoreType.DMA,
          pltpu.SemaphoreType.DMA,
      ],
  )
  def k(seg_hbm, h_hbm, z_hbm, out_hbm,
        i0, i1, i2, hb0, hb1, hb2, obuf_v, acc_sh,
        is0, is1, is2, hs0, hs1, hs2, ss0, ss1, ss2):
    idx = (i0, i1, i2)
    hbuf = (hb0, hb1, hb2)
    isem = (is0, is1, is2)
    hsem = (hs0, hs1, hs2)
    ssem = (ss0, ss1, ss2)

    cid = lax.axis_index("c")
    sid = lax.axis_index("s")
    wid = sid * _NC + cid
    nmine = jnp.where(wid < _EXTRA, _CPW + 1, _CPW)
    start = wid * _CPW + jnp.minimum(wid, _EXTRA)

    spread = lax.rem(lax.iota(jnp.int32, _L), jnp.full((_L,), _K, jnp.int32))

    def g_desc(j, b):
      row0 = (start + j) * _CHUNK
      return (pltpu.make_async_copy(seg_hbm.at[pl.ds(row0, _CHUNK)],
                                    idx[b], isem[b]),
              pltpu.make_async_copy(h_hbm.at[pl.ds(row0, _CHUNK)],
                                    hbuf[b], hsem[b]))

    def s_desc(b):
      return pltpu.make_async_copy(hbuf[b], acc_sh.at[idx[b]], ssem[b])

    def spread_idx(b):
      for i in range(_CHUNK // _L):
        sl = pl.ds(i * _L, _L)
        idx[b][sl] = idx[b][sl] * _K + spread

    @pl.when(sid == 0)
    def _():
      pltpu.sync_copy(z_hbm, acc_sh)

    plsc.subcore_barrier()

    for j in (0, 1):
      di, dh = g_desc(j, j % 3)
      di.start()
      dh.start()

    for j in range(_MAXC):
      b = j % 3
      di, dh = g_desc(j, b)

      @pl.when(j < nmine)
      def _(di=di, dh=dh, b=b):
        di.wait()
        dh.wait()
        spread_idx(b)
        s_desc(b).start(add=True)

      if j == 0:
        di2, dh2 = g_desc(2, 2)
        di2.start()
        dh2.start()
      else:
        bp = (j - 1) % 3
        di2, dh2 = g_desc(j + 2, (j + 2) % 3)

        @pl.when(j + 2 < nmine)
        def _(bp=bp, di2=di2, dh2=dh2):
          s_desc(bp).wait()
          di2.start()
          dh2.start()

    for b in range(3):
      s_desc(b).wait()

    plsc.subcore_barrier()

    @pl.when(sid == 0)
    def _():
      pltpu.sync_copy(acc_sh, obuf_v)
      pltpu.sync_copy(obuf_v, out_hbm.at[cid])

  return k(seg_ids, h, zacc)


def _readout_tc(partials, W0, b0):
  def mm(p_ref, w_ref, b_ref, o_ref):
    pooled = jnp.sum(p_ref[...], axis=(0, 2))
    o_ref[...] = lax.dot_general(
        pooled, w_ref[...], (((1,), (1,)), ((), ())),
        preferred_element_type=jnp.float32) + b_ref[...]

  return pl.pallas_call(
      mm,
      out_shape=jax.ShapeDtypeStruct((_G, _D), jnp.float32),
  )(partials, W0, b0.reshape(1, _D))


def kernel(segment_ids, h, W0, b0):
  zacc = jnp.zeros((_G * _K, _D), jnp.float32)
  partials = _segment_sum_sc(segment_ids, h, zacc)
  return _readout_tc(partials.reshape(_NC, _G, _K, _D), W0, b0)


# speedup vs baseline: 1.5236x; 1.0952x over previous
import functools

import jax
import jax.numpy as jnp
from jax import lax
from jax.experimental import pallas as pl
from jax.experimental.pallas import tpu as pltpu
from jax.experimental.pallas import tpu_sc as plsc

_N_NODES = 100000
_D = 128
_G = 64
_K = 8
_GK = _G * _K

_NC = 2
_NS = 16
_NW = _NC * _NS
_L = 16

_CHUNK = 80
_NCHUNKS = _N_NODES // _CHUNK
_CPW = _NCHUNKS // _NW
_EXTRA = _NCHUNKS - _CPW * _NW
_MAXC = _CPW + 1
_ROWS_SLICE = _GK // _NS


def _segment_sum_sc(seg_ids, h, zacc):
  mesh = plsc.VectorSubcoreMesh(core_axis_name="c", subcore_axis_name="s")

  @functools.partial(
      pl.kernel,
      out_type=jax.ShapeDtypeStruct((_NC * _GK, _D), jnp.float32),
      mesh=mesh,
      scratch_types=[
          pltpu.VMEM((_MAXC * _CHUNK,), jnp.int32),
          pltpu.VMEM((_CHUNK,), jnp.int32),
          pltpu.VMEM((_CHUNK,), jnp.int32),
          pltpu.VMEM((_CHUNK,), jnp.int32),
          pltpu.VMEM((_CHUNK, _D), jnp.float32),
          pltpu.VMEM((_CHUNK, _D), jnp.float32),
          pltpu.VMEM((_CHUNK, _D), jnp.float32),
          pltpu.VMEM_SHARED((_GK, _D), jnp.float32),
          pltpu.SemaphoreType.DMA,
          pltpu.SemaphoreType.DMA,
          pltpu.SemaphoreType.DMA,
          pltpu.SemaphoreType.DMA,
          pltpu.SemaphoreType.DMA,
          pltpu.SemaphoreType.DMA,
          pltpu.SemaphoreType.DMA,
      ],
  )
  def k(seg_hbm, h_hbm, z_hbm, out_hbm,
        ids_v, x0, x1, x2, hb0, hb1, hb2, acc_sh,
        dsem, hs0, hs1, hs2, ss0, ss1, ss2):
    sidx = (x0, x1, x2)
    hbuf = (hb0, hb1, hb2)
    hsem = (hs0, hs1, hs2)
    ssem = (ss0, ss1, ss2)

    cid = lax.axis_index("c")
    sid = lax.axis_index("s")
    wid = sid * _NC + cid
    nmine = jnp.where(wid < _EXTRA, _CPW + 1, _CPW)
    start = wid * _CPW + jnp.minimum(wid, _EXTRA)
    row_base = start * _CHUNK

    spread = lax.rem(lax.iota(jnp.int32, _L), jnp.full((_L,), _K, jnp.int32))

    def h_desc(j, b):
      return pltpu.make_async_copy(
          h_hbm.at[pl.ds(row_base + j * _CHUNK, _CHUNK)], hbuf[b], hsem[b])

    def s_desc(b):
      return pltpu.make_async_copy(hbuf[b], acc_sh.at[sidx[b]], ssem[b])

    def make_sidx(j, b):
      for i in range(_CHUNK // _L):
        v = ids_v[pl.ds(j * _CHUNK + i * _L, _L)]
        sidx[b][pl.ds(i * _L, _L)] = v * _K + spread

    pltpu.async_copy(seg_hbm.at[pl.ds(row_base, _CPW * _CHUNK)],
                     ids_v.at[pl.ds(0, _CPW * _CHUNK)], dsem)

    pltpu.sync_copy(z_hbm.at[pl.ds(sid * _ROWS_SLICE, _ROWS_SLICE)],
                    acc_sh.at[pl.ds(sid * _ROWS_SLICE, _ROWS_SLICE)])

    @pl.when(wid < _EXTRA)
    def _():
      pltpu.sync_copy(seg_hbm.at[pl.ds(row_base + _CPW * _CHUNK, _CHUNK)],
                      ids_v.at[pl.ds(_CPW * _CHUNK, _CHUNK)])

    pltpu.make_async_copy(seg_hbm.at[pl.ds(row_base, _CPW * _CHUNK)],
                          ids_v.at[pl.ds(0, _CPW * _CHUNK)], dsem).wait()

    plsc.subcore_barrier()

    for j in (0, 1):
      h_desc(j, j % 3).start()

    for j in range(_MAXC):
      b = j % 3

      @pl.when(j < nmine)
      def _(j=j, b=b):
        make_sidx(j, b)
        h_desc(j, b).wait()
        s_desc(b).start(add=True)

      if j == 0:
        h_desc(2, 2).start()
      else:
        bp = (j - 1) % 3

        @pl.when(j + 2 < nmine)
        def _(j=j, bp=bp):
          s_desc(bp).wait()
          h_desc(j + 2, (j + 2) % 3).start()

    for b in range(3):
      s_desc(b).wait()

    plsc.subcore_barrier()

    pltpu.sync_copy(
        acc_sh.at[pl.ds(sid * _ROWS_SLICE, _ROWS_SLICE)],
        out_hbm.at[pl.ds(cid * _GK + sid * _ROWS_SLICE, _ROWS_SLICE)])

  return k(seg_ids, h, zacc)


def _readout_tc(partials, W0, b0):
  def mm(p_ref, w_ref, b_ref, o_ref):
    pooled = jnp.sum(p_ref[...], axis=(0, 2))
    o_ref[...] = lax.dot_general(
        pooled, w_ref[...], (((1,), (1,)), ((), ())),
        preferred_element_type=jnp.float32) + b_ref[...]

  return pl.pallas_call(
      mm,
      out_shape=jax.ShapeDtypeStruct((_G, _D), jnp.float32),
  )(partials, W0, b0.reshape(1, _D))


def kernel(segment_ids, h, W0, b0):
  zacc = jnp.zeros((_GK, _D), jnp.float32)
  partials = _segment_sum_sc(segment_ids, h, zacc)
  return _readout_tc(partials.reshape(_NC, _G, _K, _D), W0, b0)


# speedup vs baseline: 1.5968x; 1.0480x over previous
import functools

import jax
import jax.numpy as jnp
from jax import lax
from jax.experimental import pallas as pl
from jax.experimental.pallas import tpu as pltpu
from jax.experimental.pallas import tpu_sc as plsc

_N_NODES = 100000
_D = 128
_G = 64
_K = 8
_GK = _G * _K

_NC = 2
_NS = 16
_NW = _NC * _NS
_L = 16
_NB = _D // _L

_CHUNK = 80
_NCHUNKS = _N_NODES // _CHUNK
_CPW = _NCHUNKS // _NW
_EXTRA = _NCHUNKS - _CPW * _NW
_MAXC = _CPW + 1
_GR = 4
_TRIPS = _CHUNK // _GR
_ROWS_SLICE = _GK // _NS
_OUTER = (_MAXC + 2) // 3


def _segment_sum_sc(seg_ids, h, zacc):
  mesh = plsc.VectorSubcoreMesh(core_axis_name="c", subcore_axis_name="s")

  @functools.partial(
      pl.kernel,
      out_type=jax.ShapeDtypeStruct((_NC * _GK, _D), jnp.float32),
      mesh=mesh,
      scratch_types=[
          pltpu.VMEM((_MAXC * _CHUNK,), jnp.int32),
          pltpu.VMEM((_CHUNK,), jnp.int32),
          pltpu.VMEM((_L,), jnp.int32),
          pltpu.VMEM((_CHUNK, _D), jnp.float32),
          pltpu.VMEM((_CHUNK, _D), jnp.float32),
          pltpu.VMEM((_CHUNK, _D), jnp.float32),
          pltpu.VMEM((_L, _D), jnp.float32),
          pltpu.VMEM_SHARED((_GK, _D), jnp.float32),
          pltpu.SemaphoreType.DMA,
          pltpu.SemaphoreType.DMA,
          pltpu.SemaphoreType.DMA,
          pltpu.SemaphoreType.DMA,
      ],
  )
  def k(seg_hbm, h_hbm, z_hbm, out_hbm,
        ids_v, sidx, s16, hb0, hb1, hb2, rowb, acc_sh,
        dsem, hs0, hs1, hs2):
    hbuf = (hb0, hb1, hb2)
    hsem = (hs0, hs1, hs2)

    cid = lax.axis_index("c")
    sid = lax.axis_index("s")
    wid = sid * _NC + cid
    nmine = jnp.where(wid < _EXTRA, _CPW + 1, _CPW)
    start = wid * _CPW + jnp.minimum(wid, _EXTRA)
    row_base = start * _CHUNK

    spread = lax.rem(lax.iota(jnp.int32, _L), jnp.full((_L,), _K, jnp.int32))
    zvec = jnp.zeros((_L,), jnp.float32)

    def h_desc(j, b):
      return pltpu.make_async_copy(
          h_hbm.at[pl.ds(row_base + j * _CHUNK, _CHUNK)], hbuf[b], hsem[b])

    pltpu.async_copy(seg_hbm.at[pl.ds(row_base, _CPW * _CHUNK)],
                     ids_v.at[pl.ds(0, _CPW * _CHUNK)], dsem)

    pltpu.sync_copy(z_hbm.at[pl.ds(sid * _ROWS_SLICE, _ROWS_SLICE)],
                    acc_sh.at[pl.ds(sid * _ROWS_SLICE, _ROWS_SLICE)])
    pltpu.sync_copy(z_hbm.at[pl.ds(0, _L)], rowb)

    @pl.when(wid < _EXTRA)
    def _():
      pltpu.sync_copy(seg_hbm.at[pl.ds(row_base + _CPW * _CHUNK, _CHUNK)],
                      ids_v.at[pl.ds(_CPW * _CHUNK, _CHUNK)])

    pltpu.make_async_copy(seg_hbm.at[pl.ds(row_base, _CPW * _CHUNK)],
                          ids_v.at[pl.ds(0, _CPW * _CHUNK)], dsem).wait()

    plsc.subcore_barrier()

    def flush(prev_seg):
      s16[pl.ds(0, _L)] = jnp.full((_L,), prev_seg * _K, jnp.int32) + spread
      pltpu.sync_copy(rowb, acc_sh.at[s16], add=True)
      for g in range(_GR):
        for i in range(_NB):
          rowb[g, pl.ds(i * _L, _L)] = zvec

    def accumulate(b):
      def body(t, carry):
        out = []
        for g in range(_GR):
          r = t * _GR + g
          for i in range(_NB):
            out.append(carry[g * _NB + i] + hbuf[b][r, pl.ds(i * _L, _L)])
        return tuple(out)

      init = tuple(zvec for _ in range(_GR * _NB))
      accs = lax.fori_loop(0, _TRIPS, body, init)
      for g in range(_GR):
        for i in range(_NB):
          plsc.addupdate(rowb.at[g, pl.ds(i * _L, _L)], accs[g * _NB + i])

    def boundary_chunk(jb, b):
      for i in range(_CHUNK // _L):
        v = ids_v[pl.ds(jb * _CHUNK + i * _L, _L)]
        sidx[pl.ds(i * _L, _L)] = v * _K + spread
      pltpu.sync_copy(hbuf[b], acc_sh.at[sidx], add=True)

    for j in (0, 1):
      h_desc(j, j % 3).start()

    def outer(t, prev_seg):
      for b in range(3):
        jb = t * 3 + b
        active = jb < nmine
        safe = jnp.minimum(jb, _MAXC - 1)
        vfirst = ids_v[pl.ds(safe * _CHUNK, _L)]
        vlast = ids_v[pl.ds(safe * _CHUNK + _CHUNK - _L, _L)]
        cf = vfirst[0]
        cl = vlast[_L - 1]
        uniform = cf == cl

        @pl.when(jb + 2 < nmine)
        def _(jb=jb, b=b):
          h_desc(jb + 2, (b + 2) % 3).start()

        @pl.when(active)
        def _(jb=jb, b=b, cf=cf, uniform=uniform, prev_seg=prev_seg):
          h_desc(jb, b).wait()

          @pl.when(uniform)
          def _():
            @pl.when((prev_seg >= 0) & (prev_seg != cf))
            def _():
              flush(prev_seg)

            accumulate(b)

          @pl.when(jnp.logical_not(uniform))
          def _():
            boundary_chunk(jb, b)

        prev_seg = jnp.where(active & uniform, cf, prev_seg)
      return prev_seg

    prev_seg = lax.fori_loop(0, _OUTER, outer, jnp.int32(-1))

    @pl.when(prev_seg >= 0)
    def _():
      flush(prev_seg)

    plsc.subcore_barrier()

    pltpu.sync_copy(
        acc_sh.at[pl.ds(sid * _ROWS_SLICE, _ROWS_SLICE)],
        out_hbm.at[pl.ds(cid * _GK + sid * _ROWS_SLICE, _ROWS_SLICE)])

  return k(seg_ids, h, zacc)


def _readout_tc(partials, W0, b0):
  def mm(p_ref, w_ref, b_ref, o_ref):
    pooled = jnp.sum(p_ref[...], axis=(0, 2))
    o_ref[...] = lax.dot_general(
        pooled, w_ref[...], (((1,), (1,)), ((), ())),
        preferred_element_type=jnp.float32) + b_ref[...]

  return pl.pallas_call(
      mm,
      out_shape=jax.ShapeDtypeStruct((_G, _D), jnp.float32),
  )(partials, W0, b0.reshape(1, _D))


def kernel(segment_ids, h, W0, b0):
  zacc = jnp.zeros((_GK, _D), jnp.float32)
  partials = _segment_sum_sc(segment_ids, h, zacc)
  return _readout_tc(partials.reshape(_NC, _G, _K, _D), W0, b0)


# speedup vs baseline: 1.7917x; 1.1221x over previous
import functools

import jax
import jax.numpy as jnp
from jax import lax
from jax.experimental import pallas as pl
from jax.experimental.pallas import tpu as pltpu
from jax.experimental.pallas import tpu_sc as plsc

_N_NODES = 100000
_D = 128
_G = 64
_K = 8
_GK = _G * _K

_NC = 2
_NS = 16
_NW = _NC * _NS
_L = 16
_NB = _D // _L

_N_SC = 50000
_TCBLK = 2000
_TC_OFF = _N_SC // _TCBLK
_TC_STEPS = (_N_NODES - _N_SC) // _TCBLK

_CHUNK = 80
_NCHUNKS = _N_SC // _CHUNK
_CPW = _NCHUNKS // _NW
_EXTRA = _NCHUNKS - _CPW * _NW
_MAXC = _CPW + 1
_GR = 4
_TRIPS = _CHUNK // _GR
_ROWS_SLICE = _GK // _NS
_OUTER = (_MAXC + 2) // 3


def _segment_sum_sc(seg_ids, h, zacc):
  mesh = plsc.VectorSubcoreMesh(core_axis_name="c", subcore_axis_name="s")

  @functools.partial(
      pl.kernel,
      out_type=jax.ShapeDtypeStruct((_NC * _GK, _D), jnp.float32),
      mesh=mesh,
      scratch_types=[
          pltpu.VMEM((_MAXC * _CHUNK,), jnp.int32),
          pltpu.VMEM((_CHUNK,), jnp.int32),
          pltpu.VMEM((_L,), jnp.int32),
          pltpu.VMEM((_CHUNK, _D), jnp.float32),
          pltpu.VMEM((_CHUNK, _D), jnp.float32),
          pltpu.VMEM((_CHUNK, _D), jnp.float32),
          pltpu.VMEM((_L, _D), jnp.float32),
          pltpu.VMEM_SHARED((_GK, _D), jnp.float32),
          pltpu.SemaphoreType.DMA,
          pltpu.SemaphoreType.DMA,
          pltpu.SemaphoreType.DMA,
          pltpu.SemaphoreType.DMA,
      ],
  )
  def k(seg_hbm, h_hbm, z_hbm, out_hbm,
        ids_v, sidx, s16, hb0, hb1, hb2, rowb, acc_sh,
        dsem, hs0, hs1, hs2):
    hbuf = (hb0, hb1, hb2)
    hsem = (hs0, hs1, hs2)

    cid = lax.axis_index("c")
    sid = lax.axis_index("s")
    wid = sid * _NC + cid
    nmine = jnp.where(wid < _EXTRA, _CPW + 1, _CPW)
    start = wid * _CPW + jnp.minimum(wid, _EXTRA)
    row_base = start * _CHUNK

    spread = lax.rem(lax.iota(jnp.int32, _L), jnp.full((_L,), _K, jnp.int32))
    zvec = jnp.zeros((_L,), jnp.float32)

    def h_desc(j, b):
      return pltpu.make_async_copy(
          h_hbm.at[pl.ds(row_base + j * _CHUNK, _CHUNK)], hbuf[b], hsem[b])

    pltpu.async_copy(seg_hbm.at[pl.ds(row_base, _CPW * _CHUNK)],
                     ids_v.at[pl.ds(0, _CPW * _CHUNK)], dsem)

    pltpu.sync_copy(z_hbm.at[pl.ds(sid * _ROWS_SLICE, _ROWS_SLICE)],
                    acc_sh.at[pl.ds(sid * _ROWS_SLICE, _ROWS_SLICE)])
    pltpu.sync_copy(z_hbm.at[pl.ds(0, _L)], rowb)

    @pl.when(wid < _EXTRA)
    def _():
      pltpu.sync_copy(seg_hbm.at[pl.ds(row_base + _CPW * _CHUNK, _CHUNK)],
                      ids_v.at[pl.ds(_CPW * _CHUNK, _CHUNK)])

    pltpu.make_async_copy(seg_hbm.at[pl.ds(row_base, _CPW * _CHUNK)],
                          ids_v.at[pl.ds(0, _CPW * _CHUNK)], dsem).wait()

    plsc.subcore_barrier()

    def flush(prev_seg):
      s16[pl.ds(0, _L)] = jnp.full((_L,), prev_seg * _K, jnp.int32) + spread
      pltpu.sync_copy(rowb, acc_sh.at[s16], add=True)
      for g in range(_GR):
        for i in range(_NB):
          rowb[g, pl.ds(i * _L, _L)] = zvec

    def accumulate(b):
      def body(t, carry):
        out = []
        for g in range(_GR):
          r = t * _GR + g
          for i in range(_NB):
            out.append(carry[g * _NB + i] + hbuf[b][r, pl.ds(i * _L, _L)])
        return tuple(out)

      init = tuple(zvec for _ in range(_GR * _NB))
      accs = lax.fori_loop(0, _TRIPS, body, init)
      for g in range(_GR):
        for i in range(_NB):
          plsc.addupdate(rowb.at[g, pl.ds(i * _L, _L)], accs[g * _NB + i])

    def boundary_chunk(jb, b):
      for i in range(_CHUNK // _L):
        v = ids_v[pl.ds(jb * _CHUNK + i * _L, _L)]
        sidx[pl.ds(i * _L, _L)] = v * _K + spread
      pltpu.sync_copy(hbuf[b], acc_sh.at[sidx], add=True)

    for j in (0, 1):
      h_desc(j, j % 3).start()

    def outer(t, prev_seg):
      for b in range(3):
        jb = t * 3 + b
        active = jb < nmine
        safe = jnp.minimum(jb, _MAXC - 1)
        vfirst = ids_v[pl.ds(safe * _CHUNK, _L)]
        vlast = ids_v[pl.ds(safe * _CHUNK + _CHUNK - _L, _L)]
        cf = vfirst[0]
        cl = vlast[_L - 1]
        uniform = cf == cl

        @pl.when(jb + 2 < nmine)
        def _(jb=jb, b=b):
          h_desc(jb + 2, (b + 2) % 3).start()

        @pl.when(active)
        def _(jb=jb, b=b, cf=cf, uniform=uniform, prev_seg=prev_seg):
          h_desc(jb, b).wait()

          @pl.when(uniform)
          def _():
            @pl.when((prev_seg >= 0) & (prev_seg != cf))
            def _():
              flush(prev_seg)

            accumulate(b)

          @pl.when(jnp.logical_not(uniform))
          def _():
            boundary_chunk(jb, b)

        prev_seg = jnp.where(active & uniform, cf, prev_seg)
      return prev_seg

    prev_seg = lax.fori_loop(0, _OUTER, outer, jnp.int32(-1))

    @pl.when(prev_seg >= 0)
    def _():
      flush(prev_seg)

    plsc.subcore_barrier()

    pltpu.sync_copy(
        acc_sh.at[pl.ds(sid * _ROWS_SLICE, _ROWS_SLICE)],
        out_hbm.at[pl.ds(cid * _GK + sid * _ROWS_SLICE, _ROWS_SLICE)])

  return k(seg_ids, h, zacc)


def _segment_sum_tc(seg3d, h):
  def body(ids_ref, h_ref, o_ref):
    i = pl.program_id(0)
    ids = ids_ref[0, 0, :]
    oh = (lax.broadcasted_iota(jnp.int32, (_G, _TCBLK), 0)
          == ids[None, :]).astype(jnp.float32)
    part = lax.dot_general(oh, h_ref[...], (((1,), (0,)), ((), ())),
                           preferred_element_type=jnp.float32)

    @pl.when(i == 0)
    def _():
      o_ref[...] = part

    @pl.when(i > 0)
    def _():
      o_ref[...] += part

  return pl.pallas_call(
      body,
      grid=(_TC_STEPS,),
      in_specs=[
          pl.BlockSpec((1, 1, _TCBLK), lambda i: (i + _TC_OFF, 0, 0)),
          pl.BlockSpec((_TCBLK, _D), lambda i: (i + _TC_OFF, 0)),
      ],
      out_specs=pl.BlockSpec((_G, _D), lambda i: (0, 0)),
      out_shape=jax.ShapeDtypeStruct((_G, _D), jnp.float32),
  )(seg3d, h)


def _readout_tc(partials_sc, partial_tc, W0, b0):
  def mm(p_ref, t_ref, w_ref, b_ref, o_ref):
    pooled = jnp.sum(p_ref[...], axis=(0, 2)) + t_ref[...]
    o_ref[...] = lax.dot_general(
        pooled, w_ref[...], (((1,), (1,)), ((), ())),
        preferred_element_type=jnp.float32) + b_ref[...]

  return pl.pallas_call(
      mm,
      out_shape=jax.ShapeDtypeStruct((_G, _D), jnp.float32),
  )(partials_sc, partial_tc, W0, b0.reshape(1, _D))


def kernel(segment_ids, h, W0, b0):
  zacc = jnp.zeros((_GK, _D), jnp.float32)
  partials_sc = _segment_sum_sc(segment_ids, h, zacc)
  seg3d = segment_ids.reshape(_N_NODES // _TCBLK, 1, _TCBLK)
  partial_tc = _segment_sum_tc(seg3d, h)
  return _readout_tc(partials_sc.reshape(_NC, _G, _K, _D), partial_tc, W0, b0)
